# Initial kernel scaffold; baseline (speedup 1.0000x reference)
import jax
import jax.numpy as jnp
from jax.experimental import pallas as pl


def kernel(gpt_ids, gen_len, emb_table, pos_table):
    raise NotImplementedError("write your pallas kernel here")


# speedup vs baseline: 1.1667x; 1.1667x over previous
import functools

import jax
import jax.numpy as jnp
from jax import lax
from jax.experimental import pallas as pl
from jax.experimental.pallas import tpu as pltpu
from jax.experimental.pallas import tpu_sc as plsc

VOCAB = 100000
D = 1024
B = 128
L = 32
N = B * L
NC, NS, LANES = 2, 16, 16
NW = NC * NS
ROWS_PER_W = N // NW
CHUNK = 32
NCHUNK = ROWS_PER_W // CHUNK
VPR = D // LANES


def _emb_body(ids_hbm, pidx_hbm, emb_hbm, pos_hbm, out_hbm,
              idx_v, pidx_v, pos_v, buf0, buf1, sem0, sem1):
    wid = lax.axis_index("s") * NC + lax.axis_index("c")
    base = wid * ROWS_PER_W

    pltpu.sync_copy(ids_hbm.at[pl.ds(base, ROWS_PER_W)], idx_v)
    pltpu.sync_copy(pidx_hbm, pidx_v)
    pltpu.async_copy(pos_hbm.at[pidx_v], pos_v, sem0).wait()

    bufs = (buf0, buf1)
    sems = (sem0, sem1)

    def gather(c):
        return pltpu.async_copy(
            emb_hbm.at[idx_v.at[pl.ds(c * CHUNK, CHUNK)]],
            bufs[c % 2], sems[c % 2])

    cps = [gather(0), None]
    for c in range(NCHUNK):
        buf = bufs[c % 2]
        cps[c % 2].wait()
        if c + 1 < NCHUNK:
            cps[(c + 1) % 2] = gather(c + 1)

        def col(j, carry):
            sl = pl.ds(j * LANES, LANES)
            pv = pos_v[0, sl]
            for r in range(CHUNK):
                buf[r, sl] = buf[r, sl] + pv
            return carry

        lax.fori_loop(0, VPR, col, 0)
        pltpu.sync_copy(buf, out_hbm.at[pl.ds(base + c * CHUNK, CHUNK)])


_emb_kernel = functools.partial(
    pl.kernel,
    out_type=jax.ShapeDtypeStruct((N, D), jnp.float32),
    mesh=plsc.VectorSubcoreMesh(core_axis_name="c", subcore_axis_name="s",
                                num_cores=NC, num_subcores=NS),
    scratch_types=[
        pltpu.VMEM((N // NW,), jnp.int32),
        pltpu.VMEM((1,), jnp.int32),
        pltpu.VMEM((1, D), jnp.float32),
        pltpu.VMEM((CHUNK, D), jnp.float32),
        pltpu.VMEM((CHUNK, D), jnp.float32),
        pltpu.SemaphoreType.DMA,
        pltpu.SemaphoreType.DMA,
    ],
)(_emb_body)


def kernel(gpt_ids, gen_len, emb_table, pos_table):
    ids_flat = jnp.reshape(gpt_ids, (N,)).astype(jnp.int32)
    pidx = jnp.reshape(jnp.asarray(gen_len, jnp.int32), (1,))
    flat = _emb_kernel(ids_flat, pidx, emb_table, pos_table)
    return jnp.reshape(flat, (B, L, D)), gen_len + 1


# speedup vs baseline: 1.1782x; 1.0098x over previous
import functools

import jax
import jax.numpy as jnp
from jax import lax
from jax.experimental import pallas as pl
from jax.experimental.pallas import tpu as pltpu
from jax.experimental.pallas import tpu_sc as plsc

VOCAB = 100000
D = 1024
B = 128
L = 32
N = B * L
NC, NS, LANES = 2, 16, 16
NW = NC * NS
ROWS_PER_W = N // NW
CHUNK = 32
NCHUNK = ROWS_PER_W // CHUNK
VPR = D // LANES


NBUF = 3


def _emb_body(ids_hbm, pidx_hbm, emb_hbm, pos_hbm, out_hbm,
              idx_v, pidx_v, pos_v, b0, b1, b2, g0, g1, g2, s0, s1, s2):
    bufs = (b0, b1, b2)
    gsems = (g0, g1, g2)
    ssems = (s0, s1, s2)
    wid = lax.axis_index("s") * NC + lax.axis_index("c")
    base = wid * ROWS_PER_W

    pltpu.sync_copy(ids_hbm.at[pl.ds(base, ROWS_PER_W)], idx_v)
    pltpu.sync_copy(pidx_hbm, pidx_v)
    pltpu.async_copy(pos_hbm.at[pidx_v], pos_v, gsems[0]).wait()

    def gather(c):
        return pltpu.async_copy(
            emb_hbm.at[idx_v.at[pl.ds(c * CHUNK, CHUNK)]],
            bufs[c % NBUF], gsems[c % NBUF])

    gcp = [None] * NBUF
    scp = [None] * NBUF
    for c in range(min(NBUF - 1, NCHUNK)):
        gcp[c] = gather(c)
    for c in range(NCHUNK):
        b = c % NBUF
        buf = bufs[b]
        gcp[b].wait()

        def col(j, carry):
            sl = pl.ds(j * LANES, LANES)
            pv = pos_v[0, sl]
            for r in range(CHUNK):
                buf[r, sl] = buf[r, sl] + pv
            return carry

        lax.fori_loop(0, VPR, col, 0)
        scp[b] = pltpu.async_copy(
            buf, out_hbm.at[pl.ds(base + c * CHUNK, CHUNK)], ssems[b])

        nxt = c + NBUF - 1
        if nxt < NCHUNK:
            nb = nxt % NBUF
            if scp[nb] is not None:
                scp[nb].wait()
                scp[nb] = None
            gcp[nb] = gather(nxt)
    for cp in scp:
        if cp is not None:
            cp.wait()


_emb_kernel = functools.partial(
    pl.kernel,
    out_type=jax.ShapeDtypeStruct((N, D), jnp.float32),
    mesh=plsc.VectorSubcoreMesh(core_axis_name="c", subcore_axis_name="s",
                                num_cores=NC, num_subcores=NS),
    scratch_types=[
        pltpu.VMEM((N // NW,), jnp.int32),
        pltpu.VMEM((1,), jnp.int32),
        pltpu.VMEM((1, D), jnp.float32),
        pltpu.VMEM((CHUNK, D), jnp.float32),
        pltpu.VMEM((CHUNK, D), jnp.float32),
        pltpu.VMEM((CHUNK, D), jnp.float32),
        pltpu.SemaphoreType.DMA,
        pltpu.SemaphoreType.DMA,
        pltpu.SemaphoreType.DMA,
        pltpu.SemaphoreType.DMA,
        pltpu.SemaphoreType.DMA,
        pltpu.SemaphoreType.DMA,
    ],
)(_emb_body)


def kernel(gpt_ids, gen_len, emb_table, pos_table):
    ids_flat = jnp.reshape(gpt_ids, (N,)).astype(jnp.int32)
    pidx = jnp.reshape(jnp.asarray(gen_len, jnp.int32), (1,))
    flat = _emb_kernel(ids_flat, pidx, emb_table, pos_table)
    return jnp.reshape(flat, (B, L, D)), gen_len + 1
